# baseline (speedup 1.0000x reference)
import functools

import jax
import jax.numpy as jnp
from jax import lax
from jax.experimental import pallas as pl
from jax.experimental.pallas import tpu as pltpu
from jax.experimental.pallas import tpu_sc as plsc

_N = 16384
_E = 8192
_D = 32
_BETA = 0.25


_TBLK = 256
_NBLK = _N // _TBLK
_HALF = _E // 2
_LANE = 128
_NCH = _HALF // _LANE
_ROW = 32
_NROW = _TBLK // _ROW


def _lane_argmin(m, i):
    width = _LANE
    while width > 1:
        width //= 2
        ma, mb = m[:, :width], m[:, width:2 * width]
        ia, ib = i[:, :width], i[:, width:2 * width]
        upd = (mb < ma) | ((mb == ma) & (ib < ia))
        m = jnp.where(upd, mb, ma)
        i = jnp.where(upd, ib, ia)
    return m, i


def _argmin_body(xt_ref, wt_ref, x2_ref, e2_ref, idx_ref, mm_ref):
    xt = xt_ref[...]
    mm_ref[...] = lax.dot_general(xt + xt, wt_ref[...],
                                  (((0,), (0,)), ((), ())),
                                  preferred_element_type=jnp.float32)
    e2 = e2_ref[...]
    lane = lax.broadcasted_iota(jnp.int32, (_ROW, _LANE), 1)

    for r in range(_NROW):
        r8 = r * _ROW
        x2r = x2_ref[r8:r8 + _ROW, :]

        def half_sweep(k0, x2r=x2r, r8=r8):
            m = jnp.full((_ROW, _LANE), jnp.inf, jnp.float32)
            i = jnp.zeros((_ROW, _LANE), jnp.int32)
            for k in range(_NCH):
                c0 = (k0 + k) * _LANE
                s = ((x2r + e2[:, c0:c0 + _LANE])
                     - mm_ref[r8:r8 + _ROW, c0:c0 + _LANE])
                upd = s < m
                m = jnp.minimum(m, s)
                i = jnp.where(upd, jnp.int32(k), i)
            return _lane_argmin(m, i * _LANE + lane + (k0 * _LANE))

        m0, i0 = half_sweep(0)
        m1, i1 = half_sweep(_NCH)
        m0b = m0.astype(jnp.bfloat16).astype(jnp.float32)
        ids = jnp.where(m1 < m0b, i1, i0)
        idx_ref[0, r8:r8 + _ROW, 0] = ids.reshape(_ROW)


def _argmin_call(xt, wt, x2, e2):
    out = pl.pallas_call(
        _argmin_body,
        grid=(_NBLK,),
        in_specs=[
            pl.BlockSpec((_D, _TBLK), lambda i: (0, i)),
            pl.BlockSpec((_D, _E), lambda i: (0, 0)),
            pl.BlockSpec((_TBLK, 1), lambda i: (i, 0)),
            pl.BlockSpec((1, _E), lambda i: (0, 0)),
        ],
        out_specs=pl.BlockSpec((1, _TBLK, 1), lambda i: (i, 0, 0)),
        out_shape=jax.ShapeDtypeStruct((_NBLK, _TBLK, 1), jnp.int32),
        scratch_shapes=[pltpu.VMEM((_TBLK, _E), jnp.float32)],
    )(xt, wt, x2, e2)
    return out.reshape(_N)


_NC = 2
_NS = 16
_NW = _NC * _NS
_CHUNK = _N // _NW
_ISUB = 128
_NSUB = _CHUNK // _ISUB


def _sc_body(w_hbm, idx_hbm, q_hbm, counts_hbm,
             idx2_v, rows_v, ones_v, zeros_v, hist_sh, sem):
    c = lax.axis_index("c")
    s = lax.axis_index("s")
    wid = s * _NC + c
    base = wid * _CHUNK

    for j in range(_NSUB):
        pltpu.sync_copy(idx_hbm.at[pl.ds(base + j * _ISUB, _ISUB)],
                        idx2_v.at[j])
    gathers = [
        pltpu.async_copy(w_hbm.at[idx2_v.at[j]],
                         rows_v.at[pl.ds(j * _ISUB, _ISUB)], sem)
        for j in range(_NSUB)
    ]

    @pl.when(s == 0)
    def _():
        def zf(i, t):
            zeros_v[pl.ds(i * 16, 16)] = jnp.zeros((16,), jnp.float32)
            return t
        lax.fori_loop(0, _E // 16, zf, 0)
        pltpu.sync_copy(zeros_v, hist_sh)

    def of(i, t):
        ones_v[pl.ds(i * 16, 16)] = jnp.ones((16,), jnp.float32)
        return t
    lax.fori_loop(0, _ISUB // 16, of, 0)

    for g in gathers:
        g.wait()
    pltpu.sync_copy(rows_v, q_hbm.at[pl.ds(base, _CHUNK)])

    plsc.subcore_barrier()
    for j in range(_NSUB):
        pltpu.sync_copy(ones_v, hist_sh.at[idx2_v.at[j]], add=True)
    plsc.subcore_barrier()

    @pl.when(s == 0)
    def _():
        pltpu.sync_copy(hist_sh, counts_hbm.at[c])


def _sc_call(w, idx):
    mesh = plsc.VectorSubcoreMesh(core_axis_name="c", subcore_axis_name="s")
    fn = pl.kernel(
        _sc_body,
        out_type=[
            jax.ShapeDtypeStruct((_N, _D), jnp.float32),
            jax.ShapeDtypeStruct((_NC, _E), jnp.float32),
        ],
        mesh=mesh,
        scratch_types=[
            pltpu.VMEM((_NSUB, _ISUB), jnp.int32),
            pltpu.VMEM((_CHUNK, _D), jnp.float32),
            pltpu.VMEM((_ISUB,), jnp.float32),
            pltpu.VMEM((_E,), jnp.float32),
            pltpu.VMEM_SHARED((_E,), jnp.float32),
            pltpu.SemaphoreType.DMA,
        ],
        compiler_params=pltpu.CompilerParams(use_tc_tiling_on_sc=False),
    )
    return fn(w, idx)


def _fin_body(q_ref, xt_ref, counts_ref, st_ref, loss_ref, perp_ref, sq_ref):
    i = pl.program_id(0)
    q_t = lax.transpose(q_ref[...], (1, 0))
    xt = xt_ref[...]
    dlt = q_t - xt
    st_ref[...] = xt + dlt
    part = jnp.sum(dlt * dlt)

    @pl.when(i == 0)
    def _():
        sq_ref[0] = part

    @pl.when(i > 0)
    def _():
        sq_ref[0] += part

    @pl.when(i == _NBLK - 1)
    def _():
        csum = counts_ref[0:1, :] + counts_ref[1:2, :]
        p = csum * (1.0 / _N)
        ent = -jnp.sum(p * jnp.log(p + 1e-10))
        perp_ref[...] = jnp.full((1, 1), jnp.exp(ent), jnp.float32)
        m = sq_ref[0] * (1.0 / (_N * _D))
        loss_ref[...] = jnp.full((1, 1), m + _BETA * m, jnp.float32)


def _fin_call(q, xt, counts):
    st_t, loss, perp = pl.pallas_call(
        _fin_body,
        grid=(_NBLK,),
        in_specs=[
            pl.BlockSpec((_TBLK, _D), lambda i: (i, 0)),
            pl.BlockSpec((_D, _TBLK), lambda i: (0, i)),
            pl.BlockSpec((_NC, _E), lambda i: (0, 0)),
        ],
        out_specs=[
            pl.BlockSpec((_D, _TBLK), lambda i: (0, i)),
            pl.BlockSpec((1, 1), lambda i: (0, 0)),
            pl.BlockSpec((1, 1), lambda i: (0, 0)),
        ],
        out_shape=[
            jax.ShapeDtypeStruct((_D, _N), jnp.float32),
            jax.ShapeDtypeStruct((1, 1), jnp.float32),
            jax.ShapeDtypeStruct((1, 1), jnp.float32),
        ],
        scratch_shapes=[pltpu.SMEM((1,), jnp.float32)],
    )(q, xt, counts)
    return st_t, loss.reshape(()), perp.reshape(())


def kernel(inputs, embedding_weight):
    xt = inputs.T
    wt = embedding_weight.T
    x2 = jnp.sum(inputs ** 2, axis=1, keepdims=True)
    e2 = jnp.sum(embedding_weight ** 2, axis=1).reshape(1, _E)
    idx = _argmin_call(xt, wt, x2, e2)
    q, counts = _sc_call(embedding_weight, idx)
    st_t, loss, perp = _fin_call(q, xt, counts)
    return st_t.T, loss, perp, idx[:, None]


# speedup vs baseline: 1.1165x; 1.1165x over previous
import functools

import jax
import jax.numpy as jnp
from jax import lax
from jax.experimental import pallas as pl
from jax.experimental.pallas import tpu as pltpu
from jax.experimental.pallas import tpu_sc as plsc

_N = 16384
_E = 8192
_D = 32
_BETA = 0.25


_TBLK = 256
_NBLK = _N // _TBLK
_HALF = _E // 2
_LANE = 128
_NCH = _HALF // _LANE
_ROW = 32
_NROW = _TBLK // _ROW


def _lane_argmin(m, i):
    width = _LANE
    while width > 1:
        width //= 2
        ma, mb = m[:, :width], m[:, width:2 * width]
        ia, ib = i[:, :width], i[:, width:2 * width]
        upd = (mb < ma) | ((mb == ma) & (ib < ia))
        m = jnp.where(upd, mb, ma)
        i = jnp.where(upd, ib, ia)
    return m, i


def _argmin_body(xt_ref, wt_ref, x2_ref, e2_ref, idx_ref, mm_ref):
    xt = xt_ref[...]
    mm_ref[...] = lax.dot_general(xt + xt, wt_ref[...],
                                  (((0,), (0,)), ((), ())),
                                  preferred_element_type=jnp.float32)
    e2 = e2_ref[...]
    lane = lax.broadcasted_iota(jnp.int32, (_ROW, _LANE), 1)

    for r in range(_NROW):
        r8 = r * _ROW
        x2r = x2_ref[r8:r8 + _ROW, :]

        def half_sweep(k0, x2r=x2r, r8=r8):
            m = jnp.full((_ROW, _LANE), jnp.inf, jnp.float32)
            i = jnp.zeros((_ROW, _LANE), jnp.int32)
            for k in range(_NCH):
                c0 = (k0 + k) * _LANE
                s = ((x2r + e2[:, c0:c0 + _LANE])
                     - mm_ref[r8:r8 + _ROW, c0:c0 + _LANE])
                upd = s < m
                m = jnp.minimum(m, s)
                i = jnp.where(upd, jnp.int32(k), i)
            return _lane_argmin(m, i * _LANE + lane + (k0 * _LANE))

        m0, i0 = half_sweep(0)
        m1, i1 = half_sweep(_NCH)
        m0b = m0.astype(jnp.bfloat16).astype(jnp.float32)
        ids = jnp.where(m1 < m0b, i1, i0)
        idx_ref[r8:r8 + _ROW, 0] = ids.reshape(_ROW)


def _argmin_call(xt, wt, x2, e2):
    out = pl.pallas_call(
        _argmin_body,
        grid=(_NBLK,),
        in_specs=[
            pl.BlockSpec((_D, _TBLK), lambda i: (0, i)),
            pl.BlockSpec((_D, _E), lambda i: (0, 0)),
            pl.BlockSpec((_TBLK, 1), lambda i: (i, 0)),
            pl.BlockSpec((1, _E), lambda i: (0, 0)),
        ],
        out_specs=pl.BlockSpec((_TBLK, 1), lambda i: (i, 0)),
        out_shape=jax.ShapeDtypeStruct((_N, 1), jnp.int32),
        scratch_shapes=[pltpu.VMEM((_TBLK, _E), jnp.float32)],
    )(xt, wt, x2, e2)
    return out.reshape(_N)


_NC = 2
_NS = 16
_NW = _NC * _NS
_CHUNK = _N // _NW
_ISUB = 128
_NSUB = _CHUNK // _ISUB


def _sc_body(w_hbm, idx_hbm, q_hbm, counts_hbm,
             idx2_v, rows_v, ones_v, zeros_v, hist_sh, sem):
    c = lax.axis_index("c")
    s = lax.axis_index("s")
    wid = s * _NC + c
    base = wid * _CHUNK

    for j in range(_NSUB):
        pltpu.sync_copy(idx_hbm.at[pl.ds(base + j * _ISUB, _ISUB)],
                        idx2_v.at[j])
    gathers = [
        pltpu.async_copy(w_hbm.at[idx2_v.at[j]],
                         rows_v.at[pl.ds(j * _ISUB, _ISUB)], sem)
        for j in range(_NSUB)
    ]

    @pl.when(s == 0)
    def _():
        def zf(i, t):
            zeros_v[pl.ds(i * 16, 16)] = jnp.zeros((16,), jnp.float32)
            return t
        lax.fori_loop(0, _E // 16, zf, 0)
        pltpu.sync_copy(zeros_v, hist_sh)

    def of(i, t):
        ones_v[pl.ds(i * 16, 16)] = jnp.ones((16,), jnp.float32)
        return t
    lax.fori_loop(0, _ISUB // 16, of, 0)

    for g in gathers:
        g.wait()
    pltpu.sync_copy(rows_v, q_hbm.at[pl.ds(base, _CHUNK)])

    plsc.subcore_barrier()
    for j in range(_NSUB):
        pltpu.sync_copy(ones_v, hist_sh.at[idx2_v.at[j]], add=True)
    plsc.subcore_barrier()

    @pl.when(s == 0)
    def _():
        pltpu.sync_copy(hist_sh, counts_hbm.at[c])


def _sc_call(w, idx):
    mesh = plsc.VectorSubcoreMesh(core_axis_name="c", subcore_axis_name="s")
    fn = pl.kernel(
        _sc_body,
        out_type=[
            jax.ShapeDtypeStruct((_N, _D), jnp.float32),
            jax.ShapeDtypeStruct((_NC, _E), jnp.float32),
        ],
        mesh=mesh,
        scratch_types=[
            pltpu.VMEM((_NSUB, _ISUB), jnp.int32),
            pltpu.VMEM((_CHUNK, _D), jnp.float32),
            pltpu.VMEM((_ISUB,), jnp.float32),
            pltpu.VMEM((_E,), jnp.float32),
            pltpu.VMEM_SHARED((_E,), jnp.float32),
            pltpu.SemaphoreType.DMA,
        ],
        compiler_params=pltpu.CompilerParams(use_tc_tiling_on_sc=False),
    )
    return fn(w, idx)


_FBLK = 2048
_NFB = _N // _FBLK


def _fin_body(q_ref, xt_ref, counts_ref, st_ref, loss_ref, perp_ref, sq_ref):
    i = pl.program_id(0)
    q_t = lax.transpose(q_ref[...], (1, 0))
    xt = xt_ref[...]
    dlt = q_t - xt
    st_ref[...] = xt + dlt
    part = jnp.sum(dlt * dlt)

    @pl.when(i == 0)
    def _():
        sq_ref[0] = part

    @pl.when(i > 0)
    def _():
        sq_ref[0] += part

    @pl.when(i == _NFB - 1)
    def _():
        csum = counts_ref[0:1, :] + counts_ref[1:2, :]
        p = csum * (1.0 / _N)
        ent = -jnp.sum(p * jnp.log(p + 1e-10))
        perp_ref[...] = jnp.full((1, 1), jnp.exp(ent), jnp.float32)
        m = sq_ref[0] * (1.0 / (_N * _D))
        loss_ref[...] = jnp.full((1, 1), m + _BETA * m, jnp.float32)


def _fin_call(q, xt, counts):
    st_t, loss, perp = pl.pallas_call(
        _fin_body,
        grid=(_NFB,),
        in_specs=[
            pl.BlockSpec((_FBLK, _D), lambda i: (i, 0)),
            pl.BlockSpec((_D, _FBLK), lambda i: (0, i)),
            pl.BlockSpec((_NC, _E), lambda i: (0, 0)),
        ],
        out_specs=[
            pl.BlockSpec((_D, _FBLK), lambda i: (0, i)),
            pl.BlockSpec((1, 1), lambda i: (0, 0)),
            pl.BlockSpec((1, 1), lambda i: (0, 0)),
        ],
        out_shape=[
            jax.ShapeDtypeStruct((_D, _N), jnp.float32),
            jax.ShapeDtypeStruct((1, 1), jnp.float32),
            jax.ShapeDtypeStruct((1, 1), jnp.float32),
        ],
        scratch_shapes=[pltpu.SMEM((1,), jnp.float32)],
    )(q, xt, counts)
    return st_t, loss.reshape(()), perp.reshape(())


def kernel(inputs, embedding_weight):
    xt = inputs.T
    wt = embedding_weight.T
    x2 = jnp.sum(inputs ** 2, axis=1, keepdims=True)
    e2 = jnp.sum(embedding_weight ** 2, axis=1).reshape(1, _E)
    idx = _argmin_call(xt, wt, x2, e2)
    q, counts = _sc_call(embedding_weight, idx)
    st_t, loss, perp = _fin_call(q, xt, counts)
    return st_t.T, loss, perp, idx[:, None]
